# baseline (speedup 1.0000x reference)
import functools

import jax
import jax.numpy as jnp
from jax import lax
from jax.experimental import pallas as pl
from jax.experimental.pallas import tpu as pltpu
from jax.experimental.pallas import tpu_sc as plsc

MODEL_DIM = 768
NUM_EXPERTS = 64
NUM_TOKENS = 32768
TOPK = 2
N_FLAT = NUM_TOKENS * TOPK

T_BLK = 8192

NUM_TILES = 16
CHUNK = N_FLAT // NUM_TILES
SEG = CHUNK // 16


def _router_body(x_ref, w_ref, pT_ref, i1_ref, i2_ref, w1_ref, w2_ref):
    x = x_ref[...]
    w = w_ref[...]
    logits = lax.dot_general(
        w, x, (((1,), (1,)), ((), ())), preferred_element_type=jnp.float32
    )
    m = jnp.max(logits, axis=0, keepdims=True)
    ex = jnp.exp(logits - m)
    s = jnp.sum(ex, axis=0, keepdims=True)
    probs = ex / s
    pT_ref[...] = probs

    iota = lax.broadcasted_iota(jnp.int32, probs.shape, 0)
    m1 = jnp.max(probs, axis=0, keepdims=True)
    i1 = jnp.min(
        jnp.where(probs == m1, iota, NUM_EXPERTS), axis=0, keepdims=True
    )
    masked = jnp.where(iota == i1, -1.0, probs)
    m2 = jnp.max(masked, axis=0, keepdims=True)
    i2 = jnp.min(
        jnp.where(masked == m2, iota, NUM_EXPERTS), axis=0, keepdims=True
    )
    e2 = jnp.exp(m2 - m1)
    denom = 1.0 + e2
    w1 = 1.0 / denom
    w2 = e2 / denom
    i1_ref[...] = i1.reshape(T_BLK)
    i2_ref[...] = i2.reshape(T_BLK)
    w1_ref[...] = w1.reshape(T_BLK)
    w2_ref[...] = w2.reshape(T_BLK)


def _router(inputs, W):
    grid = (NUM_TOKENS // T_BLK,)
    return pl.pallas_call(
        _router_body,
        grid=grid,
        in_specs=[
            pl.BlockSpec((T_BLK, MODEL_DIM), lambda i: (i, 0)),
            pl.BlockSpec((NUM_EXPERTS, MODEL_DIM), lambda i: (0, 0)),
        ],
        out_specs=[
            pl.BlockSpec((NUM_EXPERTS, T_BLK), lambda i: (0, i)),
            pl.BlockSpec((T_BLK,), lambda i: (i,)),
            pl.BlockSpec((T_BLK,), lambda i: (i,)),
            pl.BlockSpec((T_BLK,), lambda i: (i,)),
            pl.BlockSpec((T_BLK,), lambda i: (i,)),
        ],
        out_shape=[
            jax.ShapeDtypeStruct((NUM_EXPERTS, NUM_TOKENS), jnp.float32),
            jax.ShapeDtypeStruct((NUM_TOKENS,), jnp.int32),
            jax.ShapeDtypeStruct((NUM_TOKENS,), jnp.int32),
            jax.ShapeDtypeStruct((NUM_TOKENS,), jnp.float32),
            jax.ShapeDtypeStruct((NUM_TOKENS,), jnp.float32),
        ],
    )(inputs, W)


def _sort_body(
    idx_hbm,
    so_hbm,
    ro_hbm,
    hist_hbm,
    chunk,
    rank,
    cnt,
    seg_base,
    robuf,
    valbuf,
    totv,
    alltot,
    sh_tot,
    sh_so,
):
    t = lax.axis_index("s")
    cb = t * CHUNK
    lane = lax.iota(jnp.int32, 16)
    z16 = jnp.zeros((16,), jnp.int32)

    pltpu.sync_copy(idx_hbm.at[pl.ds(cb, CHUNK)], chunk)
    for i in range(NUM_EXPERTS * 16 // 16):
        cnt[pl.ds(i * 16, 16)] = z16

    def p1(p, carry):
        addr = lane * SEG + p
        e = plsc.load_gather(chunk, [addr])
        ca = lane * NUM_EXPERTS + e
        c = plsc.load_gather(cnt, [ca])
        plsc.store_scatter(rank, [addr], c)
        plsc.store_scatter(cnt, [ca], c + 1)
        return carry

    lax.fori_loop(0, SEG, p1, jnp.int32(0))

    for j in range(NUM_EXPERTS // 16):
        run = z16
        for l in range(16):
            seg_base[pl.ds(l * NUM_EXPERTS + j * 16, 16)] = run
            run = run + cnt[pl.ds(l * NUM_EXPERTS + j * 16, 16)]
        totv[pl.ds(j * 16, 16)] = run

    pltpu.sync_copy(totv, sh_tot.at[pl.ds(t * NUM_EXPERTS, NUM_EXPERTS)])
    plsc.subcore_barrier()
    pltpu.sync_copy(sh_tot, alltot)

    carry = jnp.int32(0)
    for j in range(NUM_EXPERTS // 16):
        grand = z16
        tbase = z16
        for tp in range(NUM_TILES):
            row = alltot[pl.ds(tp * NUM_EXPERTS + j * 16, 16)]
            grand = grand + row
            tbase = tbase + jnp.where(tp < t, row, 0)
        totv[pl.ds(j * 16, 16)] = grand
        incl = plsc.cumsum(grand)
        off = incl - grand + carry
        carry = carry + jnp.sum(grand)
        adj = off + tbase
        for l in range(16):
            sl = pl.ds(l * NUM_EXPERTS + j * 16, 16)
            seg_base[sl] = seg_base[sl] + adj

    @pl.when(t == 0)
    def _():
        pltpu.sync_copy(totv, hist_hbm)

    def p3(p, carry2):
        addr = lane * SEG + p
        e = plsc.load_gather(chunk, [addr])
        r = plsc.load_gather(rank, [addr])
        b = plsc.load_gather(seg_base, [lane * NUM_EXPERTS + e])
        ro = b + r
        plsc.store_scatter(robuf, [addr], ro)
        gi = cb + addr
        plsc.store_scatter(valbuf, [addr], lax.shift_right_logical(gi, 1))
        return carry2

    lax.fori_loop(0, SEG, p3, jnp.int32(0))

    pltpu.sync_copy(robuf, ro_hbm.at[pl.ds(cb, CHUNK)])
    pltpu.sync_copy(valbuf, sh_so.at[robuf])
    plsc.subcore_barrier()
    pltpu.sync_copy(sh_so.at[pl.ds(cb, CHUNK)], so_hbm.at[pl.ds(cb, CHUNK)])


def _trivial_body(idx_hbm, so_hbm, ro_hbm, hist_hbm, chunk, rank, cnt,
                  seg_base, robuf, valbuf, totv, alltot, sh_tot, sh_so):
    t = lax.axis_index("s")
    cb = t * CHUNK
    pltpu.sync_copy(idx_hbm.at[pl.ds(cb, CHUNK)], chunk)
    pltpu.sync_copy(chunk, ro_hbm.at[pl.ds(cb, CHUNK)])
    pltpu.sync_copy(chunk, so_hbm.at[pl.ds(cb, CHUNK)])

    @pl.when(t == 0)
    def _():
        pltpu.sync_copy(totv, hist_hbm)


def _sort_dispatch(idx_flat):
    mesh = plsc.VectorSubcoreMesh(
        core_axis_name="c", subcore_axis_name="s", num_cores=1
    )
    f = pl.kernel(
        _sort_body,
        out_type=(
            jax.ShapeDtypeStruct((N_FLAT,), jnp.int32),
            jax.ShapeDtypeStruct((N_FLAT,), jnp.int32),
            jax.ShapeDtypeStruct((NUM_EXPERTS,), jnp.int32),
        ),
        mesh=mesh,
        compiler_params=pltpu.CompilerParams(
            needs_layout_passes=False, skip_device_barrier=True
        ),
        cost_estimate=pl.CostEstimate(
            flops=500_000_000, transcendentals=0, bytes_accessed=200_000_000
        ),
        scratch_types=[
            pltpu.VMEM((CHUNK,), jnp.int32),
            pltpu.VMEM((CHUNK,), jnp.int32),
            pltpu.VMEM((16 * NUM_EXPERTS,), jnp.int32),
            pltpu.VMEM((16 * NUM_EXPERTS,), jnp.int32),
            pltpu.VMEM((CHUNK,), jnp.int32),
            pltpu.VMEM((CHUNK,), jnp.int32),
            pltpu.VMEM((NUM_EXPERTS,), jnp.int32),
            pltpu.VMEM((NUM_TILES * NUM_EXPERTS,), jnp.int32),
            pltpu.VMEM_SHARED((NUM_TILES * NUM_EXPERTS,), jnp.int32),
            pltpu.VMEM_SHARED((N_FLAT,), jnp.int32),
        ],
    )
    return f(idx_flat)


@jax.jit
def kernel(inputs, W):
    probsT, i1, i2, w1, w2 = _router(inputs, W)
    idx_flat = jnp.stack([i1, i2], axis=1).reshape(-1)
    combine_weights = jnp.stack([w1, w2], axis=1).reshape(-1)
    so, ro, hist = _sort_dispatch(idx_flat)
    return (
        so,
        ro,
        combine_weights,
        hist.astype(jnp.int64),
        probsT.T,
    )


# speedup vs baseline: 1.2655x; 1.2655x over previous
import functools

import jax
import jax.numpy as jnp
from jax import lax
from jax.experimental import pallas as pl
from jax.experimental.pallas import tpu as pltpu
from jax.experimental.pallas import tpu_sc as plsc

MODEL_DIM = 768
NUM_EXPERTS = 64
NUM_TOKENS = 32768
TOPK = 2
N_FLAT = NUM_TOKENS * TOPK

T_BLK = 4096

NUM_TILES = 16
CHUNK = N_FLAT // NUM_TILES
SEG = CHUNK // 16


def _router_body(x_ref, w_ref, pT_ref, i1_ref, i2_ref, w1_ref, w2_ref):
    x = x_ref[...]
    w = w_ref[...]
    logits = lax.dot_general(
        w, x, (((1,), (1,)), ((), ())), preferred_element_type=jnp.float32
    )
    m = jnp.max(logits, axis=0, keepdims=True)
    ex = jnp.exp(logits - m)
    s = jnp.sum(ex, axis=0, keepdims=True)
    probs = ex / s
    pT_ref[...] = probs

    iota = lax.broadcasted_iota(jnp.int32, probs.shape, 0)
    m1 = jnp.max(probs, axis=0, keepdims=True)
    i1 = jnp.min(
        jnp.where(probs == m1, iota, NUM_EXPERTS), axis=0, keepdims=True
    )
    masked = jnp.where(iota == i1, -1.0, probs)
    m2 = jnp.max(masked, axis=0, keepdims=True)
    i2 = jnp.min(
        jnp.where(masked == m2, iota, NUM_EXPERTS), axis=0, keepdims=True
    )
    e2 = jnp.exp(m2 - m1)
    denom = 1.0 + e2
    w1 = 1.0 / denom
    w2 = e2 / denom
    i1_ref[...] = i1.reshape(T_BLK)
    i2_ref[...] = i2.reshape(T_BLK)
    w1_ref[...] = w1.reshape(T_BLK)
    w2_ref[...] = w2.reshape(T_BLK)


def _router(inputs, W):
    grid = (NUM_TOKENS // T_BLK,)
    return pl.pallas_call(
        _router_body,
        grid=grid,
        in_specs=[
            pl.BlockSpec((T_BLK, MODEL_DIM), lambda i: (i, 0)),
            pl.BlockSpec((NUM_EXPERTS, MODEL_DIM), lambda i: (0, 0)),
        ],
        out_specs=[
            pl.BlockSpec((NUM_EXPERTS, T_BLK), lambda i: (0, i)),
            pl.BlockSpec((T_BLK,), lambda i: (i,)),
            pl.BlockSpec((T_BLK,), lambda i: (i,)),
            pl.BlockSpec((T_BLK,), lambda i: (i,)),
            pl.BlockSpec((T_BLK,), lambda i: (i,)),
        ],
        out_shape=[
            jax.ShapeDtypeStruct((NUM_EXPERTS, NUM_TOKENS), jnp.float32),
            jax.ShapeDtypeStruct((NUM_TOKENS,), jnp.int32),
            jax.ShapeDtypeStruct((NUM_TOKENS,), jnp.int32),
            jax.ShapeDtypeStruct((NUM_TOKENS,), jnp.float32),
            jax.ShapeDtypeStruct((NUM_TOKENS,), jnp.float32),
        ],
    )(inputs, W)


def _sort_body(
    i1_hbm,
    i2_hbm,
    w1_hbm,
    w2_hbm,
    so_hbm,
    ro_hbm,
    cw_hbm,
    hist_hbm,
    i1seg,
    i2seg,
    w1seg,
    w2seg,
    rank,
    cnt,
    seg_base,
    robuf,
    valbuf,
    cwbuf,
    totv,
    alltot,
    sh_tot,
    sh_so,
):
    t = lax.axis_index("s")
    ntok = CHUNK // 2
    tb0 = t * ntok
    cb = t * CHUNK
    lane = lax.iota(jnp.int32, 16)
    z16 = jnp.zeros((16,), jnp.int32)
    nq = SEG // 2

    pltpu.sync_copy(i1_hbm.at[pl.ds(tb0, ntok)], i1seg)
    pltpu.sync_copy(i2_hbm.at[pl.ds(tb0, ntok)], i2seg)
    pltpu.sync_copy(w1_hbm.at[pl.ds(tb0, ntok)], w1seg)
    pltpu.sync_copy(w2_hbm.at[pl.ds(tb0, ntok)], w2seg)
    for i in range(16 * NUM_EXPERTS // 16):
        cnt[pl.ds(i * 16, 16)] = z16

    def p1(q, carry):
        a = lane * nq + q
        e1 = plsc.load_gather(i1seg, [a])
        c1 = plsc.load_gather(cnt, [lane * NUM_EXPERTS + e1])
        plsc.store_scatter(rank, [lane * SEG + 2 * q], c1)
        plsc.store_scatter(cnt, [lane * NUM_EXPERTS + e1], c1 + 1)
        e2 = plsc.load_gather(i2seg, [a])
        c2 = plsc.load_gather(cnt, [lane * NUM_EXPERTS + e2])
        plsc.store_scatter(rank, [lane * SEG + 2 * q + 1], c2)
        plsc.store_scatter(cnt, [lane * NUM_EXPERTS + e2], c2 + 1)
        return carry

    lax.fori_loop(0, nq, p1, jnp.int32(0))

    for j in range(NUM_EXPERTS // 16):
        run = z16
        for l in range(16):
            seg_base[pl.ds(l * NUM_EXPERTS + j * 16, 16)] = run
            run = run + cnt[pl.ds(l * NUM_EXPERTS + j * 16, 16)]
        totv[pl.ds(j * 16, 16)] = run

    pltpu.sync_copy(totv, sh_tot.at[pl.ds(t * NUM_EXPERTS, NUM_EXPERTS)])
    plsc.subcore_barrier()
    pltpu.sync_copy(sh_tot, alltot)

    carry = jnp.int32(0)
    for j in range(NUM_EXPERTS // 16):
        grand = z16
        tbase = z16
        for tp in range(NUM_TILES):
            row = alltot[pl.ds(tp * NUM_EXPERTS + j * 16, 16)]
            grand = grand + row
            tbase = tbase + jnp.where(tp < t, row, 0)
        totv[pl.ds(j * 16, 16)] = grand
        incl = plsc.cumsum(grand)
        off = incl - grand + carry
        carry = carry + jnp.sum(grand)
        adj = off + tbase
        for l in range(16):
            sl = pl.ds(l * NUM_EXPERTS + j * 16, 16)
            seg_base[sl] = seg_base[sl] + adj

    @pl.when(t == 0)
    def _():
        pltpu.sync_copy(totv, hist_hbm)

    def p3(q, carry2):
        a = lane * nq + q
        tok = tb0 + a
        f1 = lane * SEG + 2 * q
        f2 = lane * SEG + 2 * q + 1
        e1 = plsc.load_gather(i1seg, [a])
        r1 = plsc.load_gather(rank, [f1])
        b1 = plsc.load_gather(seg_base, [lane * NUM_EXPERTS + e1])
        plsc.store_scatter(robuf, [f1], b1 + r1)
        e2 = plsc.load_gather(i2seg, [a])
        r2 = plsc.load_gather(rank, [f2])
        b2 = plsc.load_gather(seg_base, [lane * NUM_EXPERTS + e2])
        plsc.store_scatter(robuf, [f2], b2 + r2)
        plsc.store_scatter(valbuf, [f1], tok)
        plsc.store_scatter(valbuf, [f2], tok)
        g1 = plsc.load_gather(w1seg, [a])
        g2 = plsc.load_gather(w2seg, [a])
        plsc.store_scatter(cwbuf, [f1], g1)
        plsc.store_scatter(cwbuf, [f2], g2)
        return carry2

    lax.fori_loop(0, nq, p3, jnp.int32(0))

    pltpu.sync_copy(robuf, ro_hbm.at[pl.ds(cb, CHUNK)])
    pltpu.sync_copy(cwbuf, cw_hbm.at[pl.ds(cb, CHUNK)])
    pltpu.sync_copy(valbuf, sh_so.at[robuf])
    plsc.subcore_barrier()
    pltpu.sync_copy(sh_so.at[pl.ds(cb, CHUNK)], so_hbm.at[pl.ds(cb, CHUNK)])


def _sort_dispatch(i1, i2, w1, w2):
    mesh = plsc.VectorSubcoreMesh(
        core_axis_name="c", subcore_axis_name="s", num_cores=1
    )
    f = pl.kernel(
        _sort_body,
        out_type=(
            jax.ShapeDtypeStruct((N_FLAT,), jnp.int32),
            jax.ShapeDtypeStruct((N_FLAT,), jnp.int32),
            jax.ShapeDtypeStruct((N_FLAT,), jnp.float32),
            jax.ShapeDtypeStruct((NUM_EXPERTS,), jnp.int32),
        ),
        mesh=mesh,
        compiler_params=pltpu.CompilerParams(
            needs_layout_passes=False, skip_device_barrier=True
        ),
        cost_estimate=pl.CostEstimate(
            flops=4 * N_FLAT, transcendentals=0, bytes_accessed=16 * N_FLAT
        ),
        scratch_types=[
            pltpu.VMEM((CHUNK // 2,), jnp.int32),
            pltpu.VMEM((CHUNK // 2,), jnp.int32),
            pltpu.VMEM((CHUNK // 2,), jnp.float32),
            pltpu.VMEM((CHUNK // 2,), jnp.float32),
            pltpu.VMEM((CHUNK,), jnp.int32),
            pltpu.VMEM((16 * NUM_EXPERTS,), jnp.int32),
            pltpu.VMEM((16 * NUM_EXPERTS,), jnp.int32),
            pltpu.VMEM((CHUNK,), jnp.int32),
            pltpu.VMEM((CHUNK,), jnp.int32),
            pltpu.VMEM((CHUNK,), jnp.float32),
            pltpu.VMEM((NUM_EXPERTS,), jnp.int32),
            pltpu.VMEM((NUM_TILES * NUM_EXPERTS,), jnp.int32),
            pltpu.VMEM_SHARED((NUM_TILES * NUM_EXPERTS,), jnp.int32),
            pltpu.VMEM_SHARED((N_FLAT,), jnp.int32),
        ],
    )
    return f(i1, i2, w1, w2)


@jax.jit
def kernel(inputs, W):
    probsT, i1, i2, w1, w2 = _router(inputs, W)
    so, ro, combine_weights, hist = _sort_dispatch(i1, i2, w1, w2)
    return (
        so,
        ro,
        combine_weights,
        hist.astype(jnp.int64),
        probsT.T,
    )
